# baseline (speedup 1.0000x reference)
import functools

import jax
import jax.numpy as jnp
from jax import lax
from jax.experimental import pallas as pl
from jax.experimental.pallas import tpu as pltpu
from jax.experimental.pallas import tpu_sc as plsc

BATCH = 16384
NF = 100
NFIELD = 26
EMB = 16
B_CH = 2048
GRID = BATCH // B_CH

NC = 2


def _sc_body(v_hbm, fi_hbm, out_hbm, fi_v, e_v, obuf, esem, osem):
    wid = lax.axis_index("s") * NC + lax.axis_index("c")

    @pl.when(wid == 0)
    def _lookup():
        pltpu.sync_copy(fi_hbm, fi_v)
        pltpu.make_async_copy(v_hbm.at[fi_v], e_v, esem).start()
        pltpu.make_async_copy(v_hbm.at[fi_v], e_v, esem).wait()
        for f in range(NF):
            obuf[pl.ds(f * EMB, EMB)] = e_v[f, :EMB]
        pltpu.make_async_copy(obuf, out_hbm, osem).start()
        pltpu.make_async_copy(obuf, out_hbm, osem).wait()


_sc_lookup = functools.partial(
    pl.kernel,
    out_type=jax.ShapeDtypeStruct((NF * EMB,), jnp.float32),
    mesh=plsc.VectorSubcoreMesh(core_axis_name="c", subcore_axis_name="s"),
    scratch_types=[
        pltpu.VMEM((NF,), jnp.int32),
        pltpu.VMEM((NF, 128), jnp.float32),
        pltpu.VMEM((NF * EMB,), jnp.float32),
        pltpu.SemaphoreType.DMA,
        pltpu.SemaphoreType.DMA,
    ],
)(_sc_body)


def _tc_body(e_ref, x_ref, out_ref, et_ref):
    @pl.when(pl.program_id(0) == 0)
    def _build_et():
        et_ref[...] = e_ref[...].T

    for f in range(NF):
        x_row = x_ref[f:f + 1, :]
        e_col = et_ref[:, f:f + 1]
        out_ref[f] = e_col * x_row


def kernel(inputs, V, field_index):
    x_t = inputs.T
    v_pad = jnp.pad(V, ((0, 0), (0, 128 - EMB)))
    e_flat = _sc_lookup(v_pad, field_index)
    e2 = e_flat.reshape(NF, EMB)
    out_t = pl.pallas_call(
        _tc_body,
        grid=(GRID,),
        in_specs=[
            pl.BlockSpec((NF, EMB), lambda i: (0, 0)),
            pl.BlockSpec((NF, B_CH), lambda i: (0, i)),
        ],
        out_specs=pl.BlockSpec((NF, EMB, B_CH), lambda i: (0, 0, i)),
        out_shape=jax.ShapeDtypeStruct((NF, EMB, BATCH), jnp.float32),
        scratch_shapes=[pltpu.VMEM((EMB, NF), jnp.float32)],
        compiler_params=pltpu.CompilerParams(
            dimension_semantics=("arbitrary",),
        ),
    )(e2, x_t)
    return jnp.transpose(out_t, (2, 0, 1))
